# Initial kernel scaffold; baseline (speedup 1.0000x reference)
import jax
import jax.numpy as jnp
from jax.experimental import pallas as pl


def kernel(input, target):
    raise NotImplementedError("write your pallas kernel here")


# speedup vs baseline: 1.6840x; 1.6840x over previous
import jax
import jax.numpy as jnp
from jax.experimental import pallas as pl
from jax.experimental.pallas import tpu as pltpu

B = 16384
F = 64
K = int(0.3 * B)
BLK = 4096
GRID = B // BLK


def _body(x_ref, t_ref, out_ref, loss_ref):
    i = pl.program_id(0)
    d = x_ref[...] - t_ref[...]
    part = jnp.sum(d * d, axis=1) * (1.0 / F)
    loss_ref[pl.ds(i * (BLK // 128), BLK // 128), :] = part.reshape(BLK // 128, 128)

    @pl.when(i == GRID - 1)
    def _():
        loss = loss_ref[...]
        keys = jax.lax.bitcast_convert_type(loss, jnp.int32)

        def step(j, t):
            cand = t | (1 << (30 - j))
            cnt = jnp.sum((keys >= cand).astype(jnp.int32))
            return jnp.where(cnt >= K, cand, t)

        t = jax.lax.fori_loop(0, 31, step, jnp.int32(0), unroll=True)
        gt = keys > t
        c_gt = jnp.sum(gt.astype(jnp.int32))
        s_gt = jnp.sum(jnp.where(gt, loss, 0.0))
        tf = jax.lax.bitcast_convert_type(t, jnp.float32)
        out_ref[0] = (s_gt + (K - c_gt).astype(jnp.float32) * tf) * (1.0 / K)


def kernel(input, target):
    res = pl.pallas_call(
        _body,
        grid=(GRID,),
        in_specs=[
            pl.BlockSpec((BLK, F), lambda i: (i, 0)),
            pl.BlockSpec((BLK, F), lambda i: (i, 0)),
        ],
        out_specs=pl.BlockSpec(memory_space=pltpu.SMEM),
        out_shape=jax.ShapeDtypeStruct((1,), jnp.float32),
        scratch_shapes=[pltpu.VMEM((128, 128), jnp.float32)],
    )(input, target)
    return res[0]
